# baseline (speedup 1.0000x reference)
import jax
import jax.numpy as jnp
from jax import lax
from jax.experimental import pallas as pl
from jax.experimental.pallas import tpu as pltpu
from jax.experimental.pallas import tpu_sc as plsc

V, E, D, R = 10000, 320000, 128, 400
HALF = E // 2
NC, NS = 2, 16
NT = NC * NS
PER_TEC = HALF // NS
CHUNK = 80
NCHUNK = PER_TEC // CHUNK
NIDX = 4
OUTR = 624
REM = V - NS * OUTR
RELR = R // NS


def _sc_edge_body(ei_hbm, et_hbm, x_hbm, rel_hbm, acc_hbm,
                  i0, i1, i2, i3, xr0, xr1, rr0, rr1, acc_sh, rel_sh,
                  si0, si1, si2, si3, sgx0, sgx1, sgr0, sgr1, ssc0, ssc1):
    c = lax.axis_index("c")
    s = lax.axis_index("s")
    wid = c * NS + s
    iv = [i0, i1, i2, i3]
    siv = [si0, si1, si2, si3]
    xr = [xr0, xr1]
    rr = [rr0, rr1]
    sgx = [sgx0, sgx1]
    sgr = [sgr0, sgr1]
    ssc = [ssc0, ssc1]

    @pl.when(s == 0)
    def _stage_rel():
        pltpu.sync_copy(rel_hbm, rel_sh)

    zvec = jnp.zeros((16,), jnp.float32)

    def zero_row(r, _):
        for j in range(8):
            xr0[r, pl.ds(j * 16, 16)] = zvec
        return 0

    lax.fori_loop(0, CHUNK, zero_row, 0)
    row0 = s * OUTR
    for k in range(OUTR // CHUNK):
        pltpu.sync_copy(xr0, acc_sh.at[pl.ds(row0 + k * CHUNK, CHUNK), :])
    pltpu.sync_copy(xr0.at[pl.ds(0, OUTR % CHUNK), :],
                    acc_sh.at[pl.ds(row0 + OUTR - OUTR % CHUNK, OUTR % CHUNK), :])

    @pl.when(s == NS - 1)
    def _zero_rem():
        pltpu.sync_copy(xr0.at[pl.ds(0, REM), :],
                        acc_sh.at[pl.ds(NS * OUTR, REM), :])

    plsc.subcore_barrier()

    ebase = wid * PER_TEC

    def issue_idx(i, t):
        off = ebase + i * CHUNK
        pltpu.async_copy(ei_hbm.at[pl.ds(off, CHUNK)], iv[t].at[0], siv[t])
        pltpu.async_copy(et_hbm.at[pl.ds(off, CHUNK)], iv[t].at[1], siv[t])
        pltpu.async_copy(ei_hbm.at[pl.ds(E + off, CHUNK)], iv[t].at[2], siv[t])

    def wait_idx(i, t):
        off = ebase + i * CHUNK
        pltpu.make_async_copy(ei_hbm.at[pl.ds(off, CHUNK)], iv[t].at[0], siv[t]).wait()
        pltpu.make_async_copy(et_hbm.at[pl.ds(off, CHUNK)], iv[t].at[1], siv[t]).wait()
        pltpu.make_async_copy(ei_hbm.at[pl.ds(E + off, CHUNK)], iv[t].at[2], siv[t]).wait()

    use_spmem_rel = s % 2 == 0

    def issue_gather(t, b):
        pltpu.async_copy(x_hbm.at[iv[t].at[0]], xr[b], sgx[b])

        @pl.when(use_spmem_rel)
        def _rel_sp():
            pltpu.async_copy(rel_sh.at[iv[t].at[1]], rr[b], sgr[b])

        @pl.when(jnp.logical_not(use_spmem_rel))
        def _rel_hbm():
            pltpu.async_copy(rel_hbm.at[iv[t].at[1]], rr[b], sgr[b])

    def wait_gather(t, b):
        pltpu.make_async_copy(x_hbm.at[iv[t].at[0]], xr[b], sgx[b]).wait()

        @pl.when(use_spmem_rel)
        def _rel_sp():
            pltpu.make_async_copy(rel_sh.at[iv[t].at[1]], rr[b], sgr[b]).wait()

        @pl.when(jnp.logical_not(use_spmem_rel))
        def _rel_hbm():
            pltpu.make_async_copy(rel_hbm.at[iv[t].at[1]], rr[b], sgr[b]).wait()

    def issue_scatter(t, b):
        pltpu.async_copy(xr[b], acc_sh.at[iv[t].at[2]], ssc[b], add=True)

    def wait_scatter(t, b):
        pltpu.make_async_copy(xr[b], acc_sh.at[iv[t].at[2]], ssc[b]).wait()

    def compute(b):
        @plsc.parallel_loop(0, CHUNK, unroll=4)
        def mul_row(r):
            for j in range(8):
                sl = pl.ds(j * 16, 16)
                xr[b][r, sl] = xr[b][r, sl] * rr[b][r, sl]

    def step(i, b, t):
        @pl.when(i + 2 < NCHUNK)
        def _pf_idx():
            issue_idx(i + 2, (t + 2) % NIDX)

        @pl.when(i + 1 < NCHUNK)
        def _pf_rows():
            wait_idx(i + 1, (t + 1) % NIDX)

            @pl.when(i >= 1)
            def _free_buf():
                wait_scatter((t + 1) % NIDX, 1 - b)

            issue_gather((t + 1) % NIDX, 1 - b)

        wait_gather(t, b)
        compute(b)
        issue_scatter(t, b)

    issue_idx(0, 0)
    issue_idx(1, 1)
    wait_idx(0, 0)
    issue_gather(0, 0)

    def group(g, _):
        i = g * NIDX
        for k in range(NIDX):
            step(i + k, k % 2, k)
        return 0

    lax.fori_loop(0, (NCHUNK - 1) // NIDX, group, 0)
    step(NCHUNK - 1, (NCHUNK - 1) % 2, (NCHUNK - 1) % NIDX)

    wait_scatter((NCHUNK - 2) % NIDX, (NCHUNK - 2) % 2)
    wait_scatter((NCHUNK - 1) % NIDX, (NCHUNK - 1) % 2)
    plsc.subcore_barrier()

    pltpu.sync_copy(acc_sh.at[pl.ds(row0, OUTR), :],
                    acc_hbm.at[c, pl.ds(row0, OUTR), :])

    @pl.when(s == NS - 1)
    def _copy_rem():
        pltpu.sync_copy(acc_sh.at[pl.ds(NS * OUTR, REM), :],
                        acc_hbm.at[c, pl.ds(NS * OUTR, REM), :])


_sc_edge = pl.kernel(
    _sc_edge_body,
    out_type=jax.ShapeDtypeStruct((NC, V, D), jnp.float32),
    mesh=plsc.VectorSubcoreMesh(core_axis_name="c", subcore_axis_name="s"),
    scratch_types=[
        pltpu.VMEM((3, CHUNK), jnp.int32),
        pltpu.VMEM((3, CHUNK), jnp.int32),
        pltpu.VMEM((3, CHUNK), jnp.int32),
        pltpu.VMEM((3, CHUNK), jnp.int32),
        pltpu.VMEM((CHUNK, D), jnp.float32),
        pltpu.VMEM((CHUNK, D), jnp.float32),
        pltpu.VMEM((CHUNK, D), jnp.float32),
        pltpu.VMEM((CHUNK, D), jnp.float32),
        pltpu.VMEM_SHARED((V, D), jnp.float32),
        pltpu.VMEM_SHARED((R, D), jnp.float32),
        pltpu.SemaphoreType.DMA,
        pltpu.SemaphoreType.DMA,
        pltpu.SemaphoreType.DMA,
        pltpu.SemaphoreType.DMA,
        pltpu.SemaphoreType.DMA,
        pltpu.SemaphoreType.DMA,
        pltpu.SemaphoreType.DMA,
        pltpu.SemaphoreType.DMA,
        pltpu.SemaphoreType.DMA,
        pltpu.SemaphoreType.DMA,
    ],
)


def _tc_epilogue_body(acc_ref, x_ref, nn_ref, er_ref, inw_ref, outw_ref,
                      loopw_ref, looprel_ref, bias_ref, bnw_ref, bnb_ref,
                      wrel_ref, out_ref, relout_ref):
    agg = (jnp.dot(acc_ref[0], inw_ref[...], preferred_element_type=jnp.float32)
           + jnp.dot(acc_ref[1], outw_ref[...], preferred_element_type=jnp.float32))
    h = agg * nn_ref[...]
    loop_term = jnp.dot(x_ref[...] * looprel_ref[...], loopw_ref[...],
                        preferred_element_type=jnp.float32) / 3.0
    pre = h + loop_term + bias_ref[...]
    mean = jnp.mean(pre, axis=0, keepdims=True)
    cent = pre - mean
    var = jnp.mean(cent * cent, axis=0, keepdims=True)
    o = cent / jnp.sqrt(var + 1e-5) * bnw_ref[...] + bnb_ref[...]
    out_ref[...] = jnp.maximum(o, 0.0)
    relout_ref[...] = jnp.dot(er_ref[...], wrel_ref[...],
                              preferred_element_type=jnp.float32)


def kernel(x, edge_index, edge_type, node_norm, emb_rel, in_w, out_w, loop_w,
           w_rel, loop_rel, bias, bn_weight, bn_bias):
    acc = _sc_edge(edge_index.astype(jnp.int32).reshape(2 * E),
                   edge_type.astype(jnp.int32), x, emb_rel)

    out, rel_out = pl.pallas_call(
        _tc_epilogue_body,
        out_shape=(
            jax.ShapeDtypeStruct((V, D), jnp.float32),
            jax.ShapeDtypeStruct((R, D), jnp.float32),
        ),
    )(acc, x, node_norm, emb_rel, in_w, out_w, loop_w,
      loop_rel.reshape(1, D), bias.reshape(1, D), bn_weight.reshape(1, D),
      bn_bias.reshape(1, D), w_rel)
    return out, rel_out


# speedup vs baseline: 1.2387x; 1.2387x over previous
import jax
import jax.numpy as jnp
from jax import lax
from jax.experimental import pallas as pl
from jax.experimental.pallas import tpu as pltpu
from jax.experimental.pallas import tpu_sc as plsc

V, E, D, R = 10000, 320000, 128, 400
HALF = E // 2
NC, NS = 2, 16
NT = NC * NS
PER_TEC = HALF // NS
CHUNK = 80
NCHUNK = PER_TEC // CHUNK
NIDX = 4
OUTR = 624
REM = V - NS * OUTR
RELR = R // NS


def _sc_edge_body(ei_hbm, et_hbm, x_hbm, rel_hbm, acc_hbm,
                  i0, i1, i2, i3, xr0, xr1, rr0, rr1, acc_sh, rel_sh,
                  si0, si1, si2, si3, sgx0, sgx1, sgr0, sgr1, ssc0, ssc1):
    c = lax.axis_index("c")
    s = lax.axis_index("s")
    wid = c * NS + s
    iv = [i0, i1, i2, i3]
    siv = [si0, si1, si2, si3]
    xr = [xr0, xr1]
    rr = [rr0, rr1]
    sgx = [sgx0, sgx1]
    sgr = [sgr0, sgr1]
    ssc = [ssc0, ssc1]

    @pl.when(s == 0)
    def _stage_rel():
        pltpu.sync_copy(rel_hbm, rel_sh)

    zvec = jnp.zeros((16,), jnp.float32)

    def zero_row(r, _):
        for j in range(8):
            xr0[r, pl.ds(j * 16, 16)] = zvec
        return 0

    lax.fori_loop(0, CHUNK, zero_row, 0)
    row0 = s * OUTR
    for k in range(OUTR // CHUNK):
        pltpu.sync_copy(xr0, acc_sh.at[pl.ds(row0 + k * CHUNK, CHUNK), :])
    pltpu.sync_copy(xr0.at[pl.ds(0, OUTR % CHUNK), :],
                    acc_sh.at[pl.ds(row0 + OUTR - OUTR % CHUNK, OUTR % CHUNK), :])

    @pl.when(s == NS - 1)
    def _zero_rem():
        pltpu.sync_copy(xr0.at[pl.ds(0, REM), :],
                        acc_sh.at[pl.ds(NS * OUTR, REM), :])

    plsc.subcore_barrier()

    ebase = wid * PER_TEC

    def issue_idx(i, t):
        off = ebase + i * CHUNK
        pltpu.async_copy(ei_hbm.at[pl.ds(off, CHUNK)], iv[t].at[0], siv[t])
        pltpu.async_copy(et_hbm.at[pl.ds(off, CHUNK)], iv[t].at[1], siv[t])
        pltpu.async_copy(ei_hbm.at[pl.ds(E + off, CHUNK)], iv[t].at[2], siv[t])

    def wait_idx(i, t):
        off = ebase + i * CHUNK
        pltpu.make_async_copy(ei_hbm.at[pl.ds(off, CHUNK)], iv[t].at[0], siv[t]).wait()
        pltpu.make_async_copy(et_hbm.at[pl.ds(off, CHUNK)], iv[t].at[1], siv[t]).wait()
        pltpu.make_async_copy(ei_hbm.at[pl.ds(E + off, CHUNK)], iv[t].at[2], siv[t]).wait()

    def issue_gather(t, b):
        pltpu.async_copy(x_hbm.at[iv[t].at[0]], xr[b], sgx[b])
        pltpu.async_copy(rel_sh.at[iv[t].at[1]], rr[b], sgr[b])

    def wait_gather(t, b):
        pltpu.make_async_copy(x_hbm.at[iv[t].at[0]], xr[b], sgx[b]).wait()
        pltpu.make_async_copy(rel_sh.at[iv[t].at[1]], rr[b], sgr[b]).wait()

    def issue_scatter(t, b):
        pltpu.async_copy(xr[b], acc_sh.at[iv[t].at[2]], ssc[b], add=True)

    def wait_scatter(t, b):
        pltpu.make_async_copy(xr[b], acc_sh.at[iv[t].at[2]], ssc[b]).wait()

    def compute(b):
        hi = jnp.int32(-65536)

        @plsc.parallel_loop(0, CHUNK, unroll=4)
        def mul_row(r):
            for j in range(4):
                rw = rr[b][r, pl.ds(j * 16, 16)]
                ra = lax.bitcast_convert_type(rw << 16, jnp.float32)
                rb = lax.bitcast_convert_type(rw & hi, jnp.float32)
                sa = pl.ds(j * 32, 16)
                sb = pl.ds(j * 32 + 16, 16)
                xr[b][r, sa] = xr[b][r, sa] * ra
                xr[b][r, sb] = xr[b][r, sb] * rb

    def step(i, b, t):
        @pl.when(i + 2 < NCHUNK)
        def _pf_idx():
            issue_idx(i + 2, (t + 2) % NIDX)

        @pl.when(i + 1 < NCHUNK)
        def _pf_rows():
            wait_idx(i + 1, (t + 1) % NIDX)

            @pl.when(i >= 1)
            def _free_buf():
                wait_scatter((t + 1) % NIDX, 1 - b)

            issue_gather((t + 1) % NIDX, 1 - b)

        wait_gather(t, b)
        compute(b)
        issue_scatter(t, b)

    issue_idx(0, 0)
    issue_idx(1, 1)
    wait_idx(0, 0)
    issue_gather(0, 0)

    def group(g, _):
        i = g * NIDX
        for k in range(NIDX):
            step(i + k, k % 2, k)
        return 0

    lax.fori_loop(0, (NCHUNK - 1) // NIDX, group, 0)
    step(NCHUNK - 1, (NCHUNK - 1) % 2, (NCHUNK - 1) % NIDX)

    wait_scatter((NCHUNK - 2) % NIDX, (NCHUNK - 2) % 2)
    wait_scatter((NCHUNK - 1) % NIDX, (NCHUNK - 1) % 2)
    plsc.subcore_barrier()

    pltpu.sync_copy(acc_sh.at[pl.ds(row0, OUTR), :],
                    acc_hbm.at[c, pl.ds(row0, OUTR), :])

    @pl.when(s == NS - 1)
    def _copy_rem():
        pltpu.sync_copy(acc_sh.at[pl.ds(NS * OUTR, REM), :],
                        acc_hbm.at[c, pl.ds(NS * OUTR, REM), :])


_sc_edge = pl.kernel(
    _sc_edge_body,
    out_type=jax.ShapeDtypeStruct((NC, V, D), jnp.float32),
    mesh=plsc.VectorSubcoreMesh(core_axis_name="c", subcore_axis_name="s"),
    scratch_types=[
        pltpu.VMEM((3, CHUNK), jnp.int32),
        pltpu.VMEM((3, CHUNK), jnp.int32),
        pltpu.VMEM((3, CHUNK), jnp.int32),
        pltpu.VMEM((3, CHUNK), jnp.int32),
        pltpu.VMEM((CHUNK, D), jnp.float32),
        pltpu.VMEM((CHUNK, D), jnp.float32),
        pltpu.VMEM((CHUNK, D // 2), jnp.int32),
        pltpu.VMEM((CHUNK, D // 2), jnp.int32),
        pltpu.VMEM_SHARED((V, D), jnp.float32),
        pltpu.VMEM_SHARED((R, D // 2), jnp.int32),
        pltpu.SemaphoreType.DMA,
        pltpu.SemaphoreType.DMA,
        pltpu.SemaphoreType.DMA,
        pltpu.SemaphoreType.DMA,
        pltpu.SemaphoreType.DMA,
        pltpu.SemaphoreType.DMA,
        pltpu.SemaphoreType.DMA,
        pltpu.SemaphoreType.DMA,
        pltpu.SemaphoreType.DMA,
        pltpu.SemaphoreType.DMA,
    ],
)


def _tc_epilogue_body(acc_ref, x_ref, nn_ref, er_ref, inw_ref, outw_ref,
                      loopw_ref, looprel_ref, bias_ref, bnw_ref, bnb_ref,
                      wrel_ref, out_ref, relout_ref):
    agg = (jnp.dot(acc_ref[0], inw_ref[...], preferred_element_type=jnp.float32)
           + jnp.dot(acc_ref[1], outw_ref[...], preferred_element_type=jnp.float32))
    h = agg * nn_ref[...]
    loop_term = jnp.dot(x_ref[...] * looprel_ref[...], loopw_ref[...],
                        preferred_element_type=jnp.float32) / 3.0
    pre = h + loop_term + bias_ref[...]
    mean = jnp.mean(pre, axis=0, keepdims=True)
    cent = pre - mean
    var = jnp.mean(cent * cent, axis=0, keepdims=True)
    o = cent / jnp.sqrt(var + 1e-5) * bnw_ref[...] + bnb_ref[...]
    out_ref[...] = jnp.maximum(o, 0.0)
    relout_ref[...] = jnp.dot(er_ref[...], wrel_ref[...],
                              preferred_element_type=jnp.float32)


def kernel(x, edge_index, edge_type, node_norm, emb_rel, in_w, out_w, loop_w,
           w_rel, loop_rel, bias, bn_weight, bn_bias):
    rel_pairs = jax.lax.bitcast_convert_type(
        emb_rel.reshape(R, 4, 2, 16).transpose(0, 1, 3, 2)
        .reshape(R, D // 2, 2).astype(jnp.bfloat16), jnp.int32)
    acc = _sc_edge(edge_index.astype(jnp.int32).reshape(2 * E),
                   edge_type.astype(jnp.int32), x, rel_pairs)

    out, rel_out = pl.pallas_call(
        _tc_epilogue_body,
        out_shape=(
            jax.ShapeDtypeStruct((V, D), jnp.float32),
            jax.ShapeDtypeStruct((R, D), jnp.float32),
        ),
    )(acc, x, node_norm, emb_rel, in_w, out_w, loop_w,
      loop_rel.reshape(1, D), bias.reshape(1, D), bn_weight.reshape(1, D),
      bn_bias.reshape(1, D), w_rel)
    return out, rel_out
